# Initial kernel scaffold; baseline (speedup 1.0000x reference)
import jax
import jax.numpy as jnp
from jax.experimental import pallas as pl


def kernel(image):
    raise NotImplementedError("write your pallas kernel here")


# speedup vs baseline: 139.6394x; 139.6394x over previous
import functools

import jax
import jax.numpy as jnp
from jax import lax
from jax.experimental import pallas as pl
from jax.experimental.pallas import tpu as pltpu
from jax.experimental.pallas import tpu_sc as plsc

H = 512
W = 512
GH, GW = 4, 4
TH, TW = H // GH, W // GW
NBINS = 256
PIXELS = TH * TW
CLIP = 96.0
SCALE = (NBINS - 1.0) / PIXELS
RH, RW = 128, 64
NW = 32

_mesh = plsc.VectorSubcoreMesh(core_axis_name="c", subcore_axis_name="s")


def _region_origin():
    c = lax.axis_index("c")
    s = lax.axis_index("s")
    w = c * 16 + s
    t = w // 2
    half = w % 2
    r0 = (t // GW) * TH
    c0 = (t % GW) * TW + half * RW
    return w, r0, c0


def _floor_f32(x):
    ti = x.astype(jnp.int32)
    tf = ti.astype(jnp.float32)
    ti = jnp.where(tf > x, ti - 1, ti)
    return ti


@functools.partial(
    pl.kernel,
    out_type=jax.ShapeDtypeStruct((NW, NBINS), jnp.float32),
    mesh=_mesh,
    compiler_params=pltpu.CompilerParams(use_tc_tiling_on_sc=False, needs_layout_passes=False),
    scratch_types=[
        pltpu.VMEM((RH, RW * 3), jnp.float32),
        pltpu.VMEM((16, NBINS), jnp.float32),
        pltpu.VMEM((NBINS,), jnp.float32),
    ],
)
def _hist_kernel(img_hbm, part_hbm, rgb_v, h16_v, hist_v):
    w, r0, c0 = _region_origin()
    pltpu.sync_copy(img_hbm.at[pl.ds(r0, RH), pl.ds(c0 * 3, RW * 3)], rgb_v)

    zero16 = jnp.zeros((16,), jnp.float32)

    def zrow(h, _):
        def zcol(j, _):
            h16_v[h, pl.ds(j * 16, 16)] = zero16
            return 0
        return lax.fori_loop(0, 16, zcol, 0)

    lax.fori_loop(0, 16, zrow, 0)

    lanes = lax.broadcasted_iota(jnp.int32, (16,), 0)
    ones = jnp.ones((16,), jnp.float32)

    def row_body(r, _):
        rows = jnp.full((16,), r, jnp.int32)
        for g in range(4):
            colsx3 = (g * 16 + lanes) * 3
            rv = plsc.load_gather(rgb_v, [rows, colsx3])
            gv = plsc.load_gather(rgb_v, [rows, colsx3 + 1])
            bv = plsc.load_gather(rgb_v, [rows, colsx3 + 2])
            v = jnp.maximum(jnp.maximum(rv, gv), bv) / 255.0
            b = jnp.clip((v * 255.0).astype(jnp.int32), 0, NBINS - 1)
            plsc.addupdate_scatter(h16_v, [lanes, b], ones)
        return 0

    lax.fori_loop(0, RH, row_body, 0)

    def red(j, _):
        acc = zero16
        for h in range(16):
            acc = acc + h16_v[h, pl.ds(j * 16, 16)]
        hist_v[pl.ds(j * 16, 16)] = acc
        return 0

    lax.fori_loop(0, 16, red, 0)
    pltpu.sync_copy(hist_v, part_hbm.at[w])


@functools.partial(
    pl.kernel,
    out_type=jax.ShapeDtypeStruct((H, W * 3), jnp.float32),
    mesh=_mesh,
    compiler_params=pltpu.CompilerParams(use_tc_tiling_on_sc=False, needs_layout_passes=False),
    scratch_types=[
        pltpu.VMEM((RH, RW * 3), jnp.float32),
        pltpu.VMEM((RH, RW * 3), jnp.float32),
        pltpu.VMEM((NW, NBINS), jnp.float32),
        pltpu.VMEM((GH * GW * NBINS,), jnp.float32),
    ],
)
def _apply_kernel(img_hbm, part_hbm, out_hbm, rgb_v, out_v, parts_v, luts_v):
    w, r0, c0 = _region_origin()
    pltpu.sync_copy(part_hbm, parts_v)
    pltpu.sync_copy(img_hbm.at[pl.ds(r0, RH), pl.ds(c0 * 3, RW * 3)], rgb_v)

    def lut_tile(t, _):
        def sum_chunk(j, tot):
            hc = parts_v[2 * t, pl.ds(j * 16, 16)] + parts_v[2 * t + 1, pl.ds(j * 16, 16)]
            return tot + jnp.sum(jnp.minimum(hc, CLIP))

        total = lax.fori_loop(0, 16, sum_chunk, 0.0)
        epb = (float(PIXELS) - total) * (1.0 / NBINS)

        def cum_chunk(j, carry):
            hc = parts_v[2 * t, pl.ds(j * 16, 16)] + parts_v[2 * t + 1, pl.ds(j * 16, 16)]
            hc = jnp.minimum(hc, CLIP) + epb
            cs = plsc.cumsum(hc) + carry
            lut = (cs * SCALE).astype(jnp.int32).astype(jnp.float32)
            lut = jnp.clip(lut, 0.0, NBINS - 1.0)
            luts_v[pl.ds(t * NBINS + j * 16, 16)] = lut
            return carry + jnp.sum(hc)

        lax.fori_loop(0, 16, cum_chunk, 0.0)
        return 0

    lax.fori_loop(0, GH * GW, lut_tile, 0)

    lanes = lax.broadcasted_iota(jnp.int32, (16,), 0)

    xparams = []
    for g in range(4):
        cabs = (c0 + g * 16 + lanes).astype(jnp.float32)
        tx = (cabs + 0.5) * (1.0 / TW) - 0.5
        x0f = _floor_f32(tx)
        wx = tx - x0f.astype(jnp.float32)
        x0 = jnp.clip(x0f, 0, GW - 1)
        x1 = jnp.clip(x0f + 1, 0, GW - 1)
        xparams.append((x0 * NBINS, x1 * NBINS, wx, 1.0 - wx))

    def row_body(r, _):
        rows = jnp.full((16,), r, jnp.int32)
        ty = (r.astype(jnp.float32) + 0.5) * (1.0 / TH) - 0.5
        y0f = _floor_f32(ty)
        wy = ty - y0f.astype(jnp.float32)
        omwy = 1.0 - wy
        y0b = jnp.clip(y0f, 0, GH - 1) * (GW * NBINS)
        y1b = jnp.clip(y0f + 1, 0, GH - 1) * (GW * NBINS)
        for g in range(4):
            x0b, x1b, wx, omwx = xparams[g]
            colsx3 = (g * 16 + lanes) * 3
            rv = plsc.load_gather(rgb_v, [rows, colsx3])
            gv = plsc.load_gather(rgb_v, [rows, colsx3 + 1])
            bv = plsc.load_gather(rgb_v, [rows, colsx3 + 2])
            v = jnp.maximum(jnp.maximum(rv, gv), bv) / 255.0
            b = jnp.clip((v * 255.0).astype(jnp.int32), 0, NBINS - 1)
            v00 = plsc.load_gather(luts_v, [y0b + x0b + b])
            v01 = plsc.load_gather(luts_v, [y0b + x1b + b])
            v10 = plsc.load_gather(luts_v, [y1b + x0b + b])
            v11 = plsc.load_gather(luts_v, [y1b + x1b + b])
            blend = (omwy * omwx * v00 + omwy * wx * v01
                     + wy * omwx * v10 + wy * wx * v11)
            vnew = blend / (NBINS - 1.0)
            pos = v > 0.0
            ratio = jnp.where(pos, vnew / v, 0.0) * (1.0 / 255.0)
            plsc.store_scatter(out_v, [rows, colsx3],
                               jnp.where(pos, rv * ratio, vnew))
            plsc.store_scatter(out_v, [rows, colsx3 + 1],
                               jnp.where(pos, gv * ratio, vnew))
            plsc.store_scatter(out_v, [rows, colsx3 + 2],
                               jnp.where(pos, bv * ratio, vnew))
        return 0

    lax.fori_loop(0, RH, row_body, 0)
    pltpu.sync_copy(out_v, out_hbm.at[pl.ds(r0, RH), pl.ds(c0 * 3, RW * 3)])


def kernel(image):
    img2 = image.reshape(H, W * 3)
    parts = _hist_kernel(img2)
    out2 = _apply_kernel(img2, parts)
    return out2.reshape(H, W, 3)
